# Initial kernel scaffold; baseline (speedup 1.0000x reference)
import jax
import jax.numpy as jnp
from jax.experimental import pallas as pl


def kernel(x, edge_index, edge_attr, xdims, xsttype, batch, st_emb, Wl1, bl1, Wr1, g1, beta1, Wl2, bl2, Wr2, g2, beta2, Wfc1, bfc1, Wfc2, bfc2, Wfc3, bfc3):
    raise NotImplementedError("write your pallas kernel here")


# speedup vs baseline: 2.2621x; 2.2621x over previous
import functools

import jax
import jax.numpy as jnp
from jax import lax
from jax.experimental import pallas as pl
from jax.experimental.pallas import tpu as pltpu
from jax.experimental.pallas import tpu_sc as plsc

N = 100000
E = 1600000
NUM_GRAPHS = 64
H = 48
BN = 400
GRID = N // BN
CHUNK = 128
NSUB = 16
E16 = ((E + NSUB * CHUNK - 1) // (NSUB * CHUNK)) * CHUNK
EPAD = E16 * NSUB
RPS_MULT = (N // NSUB // CHUNK + 1) * CHUNK
NPAD = RPS_MULT * NSUB
RITER = RPS_MULT // CHUNK
NITER = E16 // CHUNK


def _sc_body(do_deg, *args):
  if do_deg:
    (hA, hB, hC, srcp, dstp, aggA, aggB, aggC, degO,
     acc, sidx, didx, rows, sem) = args
  else:
    (hA, hB, hC, srcp, dstp, aggA, aggB, aggC,
     acc, sidx, didx, rows, sem) = args
    degO = None
  c = lax.axis_index("c")
  s = lax.axis_index("s")

  def fill_rows(val):
    def body(r, _):
      rows[r] = jnp.full((16,), val, jnp.float32)
      return 0
    lax.fori_loop(0, CHUNK, body, 0)

  def zero_acc():
    fill_rows(0.0)
    def body(t, _):
      base = s * RPS_MULT + t * CHUNK
      pltpu.sync_copy(rows, acc.at[pl.ds(base, CHUNK)])
      return 0
    lax.fori_loop(0, RITER, body, 0)

  def gather_pass(h_hbm):
    def body(j, _):
      base = s * E16 + j * CHUNK
      pltpu.sync_copy(srcp.at[pl.ds(base, CHUNK)], sidx)
      pltpu.sync_copy(dstp.at[pl.ds(base, CHUNK)], didx)
      pltpu.async_copy(h_hbm.at[sidx], rows, sem).wait()
      pltpu.sync_copy(rows, acc.at[didx], add=True)
      return 0
    lax.fori_loop(0, NITER, body, 0)

  def deg_pass():
    fill_rows(1.0)
    def body(j, _):
      base = s * E16 + j * CHUNK
      pltpu.sync_copy(dstp.at[pl.ds(base, CHUNK)], didx)
      pltpu.sync_copy(rows, acc.at[didx], add=True)
      return 0
    lax.fori_loop(0, NITER, body, 0)

  def flush(out_hbm):
    def body(t, _):
      base = s * RPS_MULT + t * CHUNK
      pltpu.sync_copy(acc.at[pl.ds(base, CHUNK)], rows)
      pltpu.sync_copy(rows, out_hbm.at[pl.ds(base, CHUNK)])
      return 0
    lax.fori_loop(0, RITER, body, 0)

  zero_acc()
  plsc.subcore_barrier()
  pl.when(c == 0)(lambda: gather_pass(hA))
  pl.when(c == 1)(lambda: gather_pass(hB))
  plsc.subcore_barrier()
  pl.when(c == 0)(lambda: flush(aggA))
  pl.when(c == 1)(lambda: flush(aggB))
  plsc.subcore_barrier()
  zero_acc()
  plsc.subcore_barrier()
  pl.when(c == 0)(lambda: gather_pass(hC))
  if do_deg:
    pl.when(c == 1)(deg_pass)
  plsc.subcore_barrier()
  pl.when(c == 0)(lambda: flush(aggC))
  if do_deg:
    pl.when(c == 1)(lambda: flush(degO))


def _make_sc(do_deg):
  outs = [jax.ShapeDtypeStruct((NPAD, 16), jnp.float32)] * (4 if do_deg else 3)
  return pl.kernel(
      functools.partial(_sc_body, do_deg),
      out_type=outs,
      mesh=plsc.VectorSubcoreMesh(core_axis_name="c", subcore_axis_name="s"),
      compiler_params=pltpu.CompilerParams(use_tc_tiling_on_sc=False),
      scratch_types=[
          pltpu.VMEM_SHARED((NPAD, 16), jnp.float32),
          pltpu.VMEM((CHUNK,), jnp.int32),
          pltpu.VMEM((CHUNK,), jnp.int32),
          pltpu.VMEM((CHUNK, 16), jnp.float32),
          pltpu.SemaphoreType.DMA,
      ],
  )


def _k1_body(x_ref, xd_ref, xst_ref, emb_ref, h0_ref, hA_ref, hB_ref, hC_ref):
  xst = xst_ref[...]
  oh = (lax.broadcasted_iota(jnp.int32, (BN, 256), 1) == xst).astype(jnp.float32)
  emb = jnp.dot(oh, emb_ref[...], preferred_element_type=jnp.float32)
  h = jnp.concatenate(
      [x_ref[...], xd_ref[...], emb, jnp.zeros((BN, 2), jnp.float32)], axis=1)
  h0_ref[...] = h
  hA_ref[...] = h[:, 0:16]
  hB_ref[...] = h[:, 16:32]
  hC_ref[...] = h[:, 32:48]


def _combine(aA, aB, aC, dg, h, WlT, bl, WrT, g, b):
  agg = jnp.concatenate([aA[...], aB[...], aC[...]], axis=1)
  invd = 1.0 / jnp.maximum(dg[...][:, 0:1], 1.0)
  z = (jnp.dot(agg * invd, WlT[...], preferred_element_type=jnp.float32)
       + bl[...]
       + jnp.dot(h[...], WrT[...], preferred_element_type=jnp.float32))
  mu = jnp.mean(z, axis=1, keepdims=True)
  var = jnp.mean((z - mu) ** 2, axis=1, keepdims=True)
  zn = (z - mu) * lax.rsqrt(var + 1e-5) * g[...] + b[...]
  return jnp.maximum(zn, 0.0)


def _k2_body(aA, aB, aC, dg, h0, WlT, bl, WrT, g, b,
             h1_ref, oA_ref, oB_ref, oC_ref):
  h1 = _combine(aA, aB, aC, dg, h0, WlT, bl, WrT, g, b)
  h1_ref[...] = h1
  oA_ref[...] = h1[:, 0:16]
  oB_ref[...] = h1[:, 16:32]
  oC_ref[...] = h1[:, 32:48]


def _k3_body(aA, aB, aC, dg, h1, WlT, bl, WrT, g, b, bt_ref,
             W1T, b1, W2T, b2, W3T, b3, out_ref, ssum, smax, scnt):
  i = pl.program_id(0)

  @pl.when(i == 0)
  def _init():
    ssum[...] = jnp.zeros((NUM_GRAPHS, H), jnp.float32)
    smax[...] = jnp.full((NUM_GRAPHS, H), -jnp.inf, jnp.float32)
    scnt[...] = jnp.zeros((NUM_GRAPHS, 1), jnp.float32)

  h2 = _combine(aA, aB, aC, dg, h1, WlT, bl, WrT, g, b)
  bt = bt_ref[...]
  m = bt == lax.broadcasted_iota(jnp.int32, (BN, NUM_GRAPHS), 1)
  mf = m.astype(jnp.float32)
  ssum[...] += lax.dot_general(mf, h2, (((0,), (0,)), ((), ())),
                               preferred_element_type=jnp.float32)
  scnt[...] += jnp.sum(mf, axis=0).reshape(NUM_GRAPHS, 1)
  for gi in range(NUM_GRAPHS):
    v = jnp.max(jnp.where(m[:, gi:gi + 1], h2, -jnp.inf), axis=0)
    smax[gi:gi + 1, :] = jnp.maximum(smax[gi:gi + 1, :], v.reshape(1, H))

  @pl.when(i == GRID - 1)
  def _fin():
    mean = ssum[...] / jnp.maximum(scnt[...], 1.0)
    zz = jnp.concatenate([mean, smax[...]], axis=1)
    zz = jnp.maximum(jnp.dot(zz, W1T[...], preferred_element_type=jnp.float32) + b1[...], 0.0)
    zz = jnp.maximum(jnp.dot(zz, W2T[...], preferred_element_type=jnp.float32) + b2[...], 0.0)
    zz = jnp.maximum(jnp.dot(zz, W3T[...], preferred_element_type=jnp.float32) + b3[...], 0.0)
    out_ref[...] = zz


def _row_spec(w):
  return pl.BlockSpec((BN, w), lambda i: (i, 0))


def _full_spec(shape):
  return pl.BlockSpec(shape, lambda i: tuple(0 for _ in shape))


def kernel(x, edge_index, edge_attr, xdims, xsttype, batch, st_emb,
           Wl1, bl1, Wr1, g1, beta1, Wl2, bl2, Wr2, g2, beta2,
           Wfc1, bfc1, Wfc2, bfc2, Wfc3, bfc3):
  f32 = jnp.float32
  srcp = jnp.concatenate([edge_index[0], jnp.zeros((EPAD - E,), jnp.int32)])
  dstp = jnp.concatenate([edge_index[1], jnp.full((EPAD - E,), N, jnp.int32)])
  Wl1T = jnp.pad(Wl1, ((0, 0), (0, 2))).T
  Wr1T = jnp.pad(Wr1, ((0, 0), (0, 2))).T
  Wl2T = Wl2.T
  Wr2T = Wr2.T
  row = lambda v: v.reshape(1, -1)

  k1 = pl.pallas_call(
      _k1_body,
      grid=(GRID,),
      in_specs=[_row_spec(32), _row_spec(2), _row_spec(1), _full_spec((256, 12))],
      out_specs=[_row_spec(48), _row_spec(16), _row_spec(16), _row_spec(16)],
      out_shape=[jax.ShapeDtypeStruct((N, 48), f32)] +
                [jax.ShapeDtypeStruct((N, 16), f32)] * 3,
  )
  h0, hA, hB, hC = k1(x, xdims, xsttype.reshape(N, 1), st_emb)

  sc1 = _make_sc(True)
  aggA, aggB, aggC, deg = sc1(hA, hB, hC, srcp, dstp)

  wspec = [_full_spec((48, 48)), _full_spec((1, 48)), _full_spec((48, 48)),
           _full_spec((1, 48)), _full_spec((1, 48))]
  k2 = pl.pallas_call(
      _k2_body,
      grid=(GRID,),
      in_specs=[_row_spec(16)] * 3 + [_row_spec(16), _row_spec(48)] + wspec,
      out_specs=[_row_spec(48), _row_spec(16), _row_spec(16), _row_spec(16)],
      out_shape=[jax.ShapeDtypeStruct((N, 48), f32)] +
                [jax.ShapeDtypeStruct((N, 16), f32)] * 3,
  )
  h1, h1A, h1B, h1C = k2(aggA, aggB, aggC, deg, h0,
                         Wl1T, row(bl1), Wr1T, row(g1), row(beta1))

  sc2 = _make_sc(False)
  agg2A, agg2B, agg2C = sc2(h1A, h1B, h1C, srcp, dstp)

  k3 = pl.pallas_call(
      _k3_body,
      grid=(GRID,),
      in_specs=([_row_spec(16)] * 3 + [_row_spec(16), _row_spec(48)] + wspec +
                [_row_spec(1),
                 _full_spec((96, 50)), _full_spec((1, 50)),
                 _full_spec((50, 50)), _full_spec((1, 50)),
                 _full_spec((50, 10)), _full_spec((1, 10))]),
      out_specs=pl.BlockSpec((NUM_GRAPHS, 10), lambda i: (0, 0)),
      out_shape=jax.ShapeDtypeStruct((NUM_GRAPHS, 10), f32),
      scratch_shapes=[pltpu.VMEM((NUM_GRAPHS, H), f32),
                      pltpu.VMEM((NUM_GRAPHS, H), f32),
                      pltpu.VMEM((NUM_GRAPHS, 1), f32)],
  )
  out = k3(agg2A, agg2B, agg2C, deg, h1,
           Wl2T, row(bl2), Wr2T, row(g2), row(beta2), batch.reshape(N, 1),
           Wfc1.T, row(bfc1), Wfc2.T, row(bfc2), Wfc3.T, row(bfc3))
  return out


# speedup vs baseline: 4.0424x; 1.7870x over previous
import functools

import jax
import jax.numpy as jnp
from jax import lax
from jax.experimental import pallas as pl
from jax.experimental.pallas import tpu as pltpu
from jax.experimental.pallas import tpu_sc as plsc

N = 100000
E = 1600000
NUM_GRAPHS = 64
H = 48
BN = 400
GRID = N // BN
CHUNK = 128
NSUB = 16
E16 = 102400
EPAD = E16 * NSUB
NITER = E16 // CHUNK
KB = 1
GB = KB * CHUNK
GROUPS = NITER // KB
DEPTH = 4
RPS = 6400
NPAD = RPS * NSUB
FC = 256
FITER = RPS // FC


def _sc_body(do_deg, *args):
  if do_deg:
    (hA, hB, hC, srcp, dst2d, aggA, aggB, aggC, degO,
     acc, sidx, didx, rows, fbuf, *sems) = args
  else:
    (hA, hB, hC, srcp, dst2d, aggA, aggB, aggC,
     acc, sidx, didx, rows, fbuf, *sems) = args
    degO = None
  c = lax.axis_index("c")
  s = lax.axis_index("s")
  isem = sems[:DEPTH]
  gsem = sems[DEPTH:]

  def fill_fbuf(val):
    def body(r, _):
      fbuf[r] = jnp.full((16,), val, jnp.float32)
      return 0
    lax.fori_loop(0, FC, body, 0)

  def zero_acc():
    fill_fbuf(0.0)
    def body(t, _):
      base = s * RPS + t * FC
      pltpu.sync_copy(fbuf, acc.at[pl.ds(base, FC)])
      return 0
    lax.fori_loop(0, FITER, body, 0)

  def idx_args(g, p):
    row0 = s * NITER + g * KB
    return [(srcp.at[pl.ds(s * E16 + g * GB, GB)], sidx.at[p], isem[p]),
            (dst2d.at[pl.ds(row0, KB)], didx.at[p], isem[p])]

  def load_idx(g, p):
    for a in idx_args(g, p):
      pltpu.async_copy(*a)

  def drain_idx(g, p):
    for a in idx_args(g, p):
      pltpu.make_async_copy(*a).wait()

  def gat_args(h_hbm, p, b):
    return (h_hbm.at[sidx.at[p, pl.ds(b * CHUNK, CHUNK)]], rows.at[p, b],
            gsem[p])

  def issue_gathers(h_hbm, p):
    for b in range(KB):
      pltpu.async_copy(*gat_args(h_hbm, p, b))

  def drain_gathers(h_hbm, p):
    for b in range(KB):
      pltpu.make_async_copy(*gat_args(h_hbm, p, b)).wait()

  def scatters(p):
    for b in range(KB):
      pltpu.sync_copy(rows.at[p, b], acc.at[didx.at[p, b]], add=True)

  def edge_pass(h_hbm):
    if h_hbm is None:
      for p in range(DEPTH):
        for b in range(KB):
          def body(r, _, p=p, b=b):
            rows[p, b, r] = jnp.full((16,), 1.0, jnp.float32)
            return 0
          lax.fori_loop(0, CHUNK, body, 0)
    load_idx(0, 0)
    drain_idx(0, 0)
    if h_hbm is not None:
      issue_gathers(h_hbm, 0)
    for p in range(1, DEPTH):
      load_idx(p, p)

    def outer(G, _):
      for p in range(DEPTH):
        g = DEPTH * G + p
        q = (p + 1) % DEPTH

        @pl.when(g + 1 < GROUPS)
        def _():
          drain_idx(g + 1, q)
          if h_hbm is not None:
            issue_gathers(h_hbm, q)

        if h_hbm is not None:
          drain_gathers(h_hbm, p)
        scatters(p)

        @pl.when(g + DEPTH < GROUPS)
        def _():
          load_idx(g + DEPTH, p)
      return 0
    lax.fori_loop(0, GROUPS // DEPTH, outer, 0)

  def gather_pass(h_hbm):
    edge_pass(h_hbm)

  def deg_pass():
    edge_pass(None)

  def flush(out_hbm):
    def body(t, _):
      base = s * RPS + t * FC
      pltpu.sync_copy(acc.at[pl.ds(base, FC)], fbuf)
      pltpu.sync_copy(fbuf, out_hbm.at[pl.ds(base, FC)])
      return 0
    lax.fori_loop(0, FITER, body, 0)

  zero_acc()
  plsc.subcore_barrier()
  pl.when(c == 0)(lambda: gather_pass(hA))
  pl.when(c == 1)(lambda: gather_pass(hB))
  plsc.subcore_barrier()
  pl.when(c == 0)(lambda: flush(aggA))
  pl.when(c == 1)(lambda: flush(aggB))
  plsc.subcore_barrier()
  zero_acc()
  plsc.subcore_barrier()
  pl.when(c == 0)(lambda: gather_pass(hC))
  if do_deg:
    pl.when(c == 1)(deg_pass)
  plsc.subcore_barrier()
  pl.when(c == 0)(lambda: flush(aggC))
  if do_deg:
    pl.when(c == 1)(lambda: flush(degO))


def _make_sc(do_deg):
  outs = [jax.ShapeDtypeStruct((NPAD, 16), jnp.float32)] * (4 if do_deg else 3)
  return pl.kernel(
      functools.partial(_sc_body, do_deg),
      out_type=outs,
      mesh=plsc.VectorSubcoreMesh(core_axis_name="c", subcore_axis_name="s"),
      compiler_params=pltpu.CompilerParams(use_tc_tiling_on_sc=False),
      scratch_types=[
          pltpu.VMEM_SHARED((NPAD, 16), jnp.float32),
          pltpu.VMEM((DEPTH, GB), jnp.int32),
          pltpu.VMEM((DEPTH, KB, CHUNK), jnp.int32),
          pltpu.VMEM((DEPTH, KB, CHUNK, 16), jnp.float32),
          pltpu.VMEM((FC, 16), jnp.float32),
      ] + [pltpu.SemaphoreType.DMA] * (2 * DEPTH),
  )


def _k1_body(x_ref, xd_ref, xst_ref, emb_ref, h0_ref, hA_ref, hB_ref, hC_ref):
  xst = xst_ref[...]
  oh = (lax.broadcasted_iota(jnp.int32, (BN, 256), 1) == xst).astype(jnp.float32)
  emb = jnp.dot(oh, emb_ref[...], preferred_element_type=jnp.float32)
  h = jnp.concatenate(
      [x_ref[...], xd_ref[...], emb, jnp.zeros((BN, 2), jnp.float32)], axis=1)
  h0_ref[...] = h
  hA_ref[...] = h[:, 0:16]
  hB_ref[...] = h[:, 16:32]
  hC_ref[...] = h[:, 32:48]


def _combine(aA, aB, aC, dg, h, WlT, bl, WrT, g, b):
  agg = jnp.concatenate([aA[...], aB[...], aC[...]], axis=1)
  invd = 1.0 / jnp.maximum(dg[...][:, 0:1], 1.0)
  z = (jnp.dot(agg * invd, WlT[...], preferred_element_type=jnp.float32)
       + bl[...]
       + jnp.dot(h[...], WrT[...], preferred_element_type=jnp.float32))
  mu = jnp.mean(z, axis=1, keepdims=True)
  var = jnp.mean((z - mu) ** 2, axis=1, keepdims=True)
  zn = (z - mu) * lax.rsqrt(var + 1e-5) * g[...] + b[...]
  return jnp.maximum(zn, 0.0)


def _k2_body(aA, aB, aC, dg, h0, WlT, bl, WrT, g, b,
             h1_ref, oA_ref, oB_ref, oC_ref):
  h1 = _combine(aA, aB, aC, dg, h0, WlT, bl, WrT, g, b)
  h1_ref[...] = h1
  oA_ref[...] = h1[:, 0:16]
  oB_ref[...] = h1[:, 16:32]
  oC_ref[...] = h1[:, 32:48]


def _k3_body(aA, aB, aC, dg, h1, WlT, bl, WrT, g, b, bt_ref,
             W1T, b1, W2T, b2, W3T, b3, out_ref, ssum, smax, scnt):
  i = pl.program_id(0)

  @pl.when(i == 0)
  def _init():
    ssum[...] = jnp.zeros((NUM_GRAPHS, H), jnp.float32)
    smax[...] = jnp.full((NUM_GRAPHS, H), -jnp.inf, jnp.float32)
    scnt[...] = jnp.zeros((NUM_GRAPHS, 1), jnp.float32)

  h2 = _combine(aA, aB, aC, dg, h1, WlT, bl, WrT, g, b)
  bt = bt_ref[...]
  m = bt == lax.broadcasted_iota(jnp.int32, (BN, NUM_GRAPHS), 1)
  mf = m.astype(jnp.float32)
  ssum[...] += lax.dot_general(mf, h2, (((0,), (0,)), ((), ())),
                               preferred_element_type=jnp.float32)
  scnt[...] += jnp.sum(mf, axis=0).reshape(NUM_GRAPHS, 1)
  for gi in range(NUM_GRAPHS):
    v = jnp.max(jnp.where(m[:, gi:gi + 1], h2, -jnp.inf), axis=0)
    smax[gi:gi + 1, :] = jnp.maximum(smax[gi:gi + 1, :], v.reshape(1, H))

  @pl.when(i == GRID - 1)
  def _fin():
    mean = ssum[...] / jnp.maximum(scnt[...], 1.0)
    zz = jnp.concatenate([mean, smax[...]], axis=1)
    zz = jnp.maximum(jnp.dot(zz, W1T[...], preferred_element_type=jnp.float32) + b1[...], 0.0)
    zz = jnp.maximum(jnp.dot(zz, W2T[...], preferred_element_type=jnp.float32) + b2[...], 0.0)
    zz = jnp.maximum(jnp.dot(zz, W3T[...], preferred_element_type=jnp.float32) + b3[...], 0.0)
    out_ref[...] = zz


def _row_spec(w):
  return pl.BlockSpec((BN, w), lambda i: (i, 0))


def _full_spec(shape):
  return pl.BlockSpec(shape, lambda i: tuple(0 for _ in shape))


def kernel(x, edge_index, edge_attr, xdims, xsttype, batch, st_emb,
           Wl1, bl1, Wr1, g1, beta1, Wl2, bl2, Wr2, g2, beta2,
           Wfc1, bfc1, Wfc2, bfc2, Wfc3, bfc3):
  f32 = jnp.float32
  srcp = jnp.concatenate([edge_index[0], jnp.zeros((EPAD - E,), jnp.int32)])
  dstp = jnp.concatenate([edge_index[1], jnp.full((EPAD - E,), N, jnp.int32)])
  dst2d = dstp.reshape(EPAD // CHUNK, CHUNK)
  Wl1T = jnp.pad(Wl1, ((0, 0), (0, 2))).T
  Wr1T = jnp.pad(Wr1, ((0, 0), (0, 2))).T
  Wl2T = Wl2.T
  Wr2T = Wr2.T
  row = lambda v: v.reshape(1, -1)

  k1 = pl.pallas_call(
      _k1_body,
      grid=(GRID,),
      in_specs=[_row_spec(32), _row_spec(2), _row_spec(1), _full_spec((256, 12))],
      out_specs=[_row_spec(48), _row_spec(16), _row_spec(16), _row_spec(16)],
      out_shape=[jax.ShapeDtypeStruct((N, 48), f32)] +
                [jax.ShapeDtypeStruct((N, 16), f32)] * 3,
  )
  h0, hA, hB, hC = k1(x, xdims, xsttype.reshape(N, 1), st_emb)

  sc1 = _make_sc(True)
  aggA, aggB, aggC, deg = sc1(hA, hB, hC, srcp, dst2d)

  wspec = [_full_spec((48, 48)), _full_spec((1, 48)), _full_spec((48, 48)),
           _full_spec((1, 48)), _full_spec((1, 48))]
  k2 = pl.pallas_call(
      _k2_body,
      grid=(GRID,),
      in_specs=[_row_spec(16)] * 3 + [_row_spec(16), _row_spec(48)] + wspec,
      out_specs=[_row_spec(48), _row_spec(16), _row_spec(16), _row_spec(16)],
      out_shape=[jax.ShapeDtypeStruct((N, 48), f32)] +
                [jax.ShapeDtypeStruct((N, 16), f32)] * 3,
  )
  h1, h1A, h1B, h1C = k2(aggA, aggB, aggC, deg, h0,
                         Wl1T, row(bl1), Wr1T, row(g1), row(beta1))

  sc2 = _make_sc(False)
  agg2A, agg2B, agg2C = sc2(h1A, h1B, h1C, srcp, dst2d)

  k3 = pl.pallas_call(
      _k3_body,
      grid=(GRID,),
      in_specs=([_row_spec(16)] * 3 + [_row_spec(16), _row_spec(48)] + wspec +
                [_row_spec(1),
                 _full_spec((96, 50)), _full_spec((1, 50)),
                 _full_spec((50, 50)), _full_spec((1, 50)),
                 _full_spec((50, 10)), _full_spec((1, 10))]),
      out_specs=pl.BlockSpec((NUM_GRAPHS, 10), lambda i: (0, 0)),
      out_shape=jax.ShapeDtypeStruct((NUM_GRAPHS, 10), f32),
      scratch_shapes=[pltpu.VMEM((NUM_GRAPHS, H), f32),
                      pltpu.VMEM((NUM_GRAPHS, H), f32),
                      pltpu.VMEM((NUM_GRAPHS, 1), f32)],
  )
  out = k3(agg2A, agg2B, agg2C, deg, h1,
           Wl2T, row(bl2), Wr2T, row(g2), row(beta2), batch.reshape(N, 1),
           Wfc1.T, row(bfc1), Wfc2.T, row(bfc2), Wfc3.T, row(bfc3))
  return out


# speedup vs baseline: 4.2907x; 1.0614x over previous
import functools

import jax
import jax.numpy as jnp
from jax import lax
from jax.experimental import pallas as pl
from jax.experimental.pallas import tpu as pltpu
from jax.experimental.pallas import tpu_sc as plsc

N = 100000
E = 1600000
NUM_GRAPHS = 64
H = 48
BN = 400
GRID = N // BN
CHUNK = 128
NSUB = 16
E16 = 102400
EPAD = E16 * NSUB
NITER = E16 // CHUNK
GROUPS = NITER
DEPTH = 4
IDXD = 2 * DEPTH
RPS = 6400
NPAD = RPS * NSUB
FC = 256
FITER = RPS // FC


def _sc_body(do_deg, *args):
  if do_deg:
    (hA, hB, hC, srcp, dst2d, aggA, aggB, aggC, degO,
     acc, sidx, didx, rows, fbuf, *sems) = args
  else:
    (hA, hB, hC, srcp, dst2d, aggA, aggB, aggC,
     acc, sidx, didx, rows, fbuf, *sems) = args
    degO = None
  c = lax.axis_index("c")
  s = lax.axis_index("s")
  isem = sems[:IDXD]
  gsem = sems[IDXD:IDXD + DEPTH]
  ssem = sems[IDXD + DEPTH:]

  def fill_fbuf(val):
    def body(r, _):
      fbuf[r] = jnp.full((16,), val, jnp.float32)
      return 0
    lax.fori_loop(0, FC, body, 0)

  def zero_acc():
    fill_fbuf(0.0)
    def body(t, _):
      base = s * RPS + t * FC
      pltpu.sync_copy(fbuf, acc.at[pl.ds(base, FC)])
      return 0
    lax.fori_loop(0, FITER, body, 0)

  def idx_args(g, p):
    return [(srcp.at[pl.ds(s * E16 + g * CHUNK, CHUNK)], sidx.at[p], isem[p]),
            (dst2d.at[pl.ds(s * NITER + g, 1)], didx.at[p], isem[p])]

  def load_idx(g, p):
    for a in idx_args(g, p):
      pltpu.async_copy(*a)

  def drain_idx(g, p):
    for a in idx_args(g, p):
      pltpu.make_async_copy(*a).wait()

  def gat_args(h_hbm, p, ip):
    return (h_hbm.at[sidx.at[ip]], rows.at[p], gsem[p])

  def sca_args(p, ip):
    return (rows.at[p], acc.at[didx.at[ip, 0]], ssem[p])

  def edge_pass(h_hbm):
    if h_hbm is None:
      for p in range(DEPTH):
        def body(r, _, p=p):
          rows[p, r] = jnp.full((16,), 1.0, jnp.float32)
          return 0
        lax.fori_loop(0, CHUNK, body, 0)
    for g0 in range(DEPTH):
      load_idx(g0, g0)
    for g0 in range(2):
      drain_idx(g0, g0)
      if h_hbm is not None:
        pltpu.async_copy(*gat_args(h_hbm, g0, g0))

    def outer(G, _):
      for u in range(IDXD):
        g = IDXD * G + u
        r = u % DEPTH
        q2 = (u + 2) % DEPTH
        i2 = (u + 2) % IDXD
        i4 = (u + 4) % IDXD

        @pl.when(jnp.logical_and(g + 2 < GROUPS, g >= 2))
        def _():
          pltpu.make_async_copy(*sca_args(q2, 0)).wait()

        @pl.when(g + 2 < GROUPS)
        def _():
          drain_idx(g + 2, i2)
          if h_hbm is not None:
            pltpu.async_copy(*gat_args(h_hbm, q2, i2))

        if h_hbm is not None:
          pltpu.make_async_copy(*gat_args(h_hbm, r, u)).wait()
        pltpu.async_copy(*sca_args(r, u), add=True)

        @pl.when(g + 4 < GROUPS)
        def _():
          load_idx(g + 4, i4)
      return 0
    lax.fori_loop(0, GROUPS // IDXD, outer, 0)
    for p in range(DEPTH):
      pltpu.make_async_copy(*sca_args(p, 0)).wait()

  def gather_pass(h_hbm):
    edge_pass(h_hbm)

  def deg_pass():
    edge_pass(None)

  def flush(out_hbm):
    def body(t, _):
      base = s * RPS + t * FC
      pltpu.sync_copy(acc.at[pl.ds(base, FC)], fbuf)
      pltpu.sync_copy(fbuf, out_hbm.at[pl.ds(base, FC)])
      return 0
    lax.fori_loop(0, FITER, body, 0)

  zero_acc()
  plsc.subcore_barrier()
  pl.when(c == 0)(lambda: gather_pass(hA))
  pl.when(c == 1)(lambda: gather_pass(hB))
  plsc.subcore_barrier()
  pl.when(c == 0)(lambda: flush(aggA))
  pl.when(c == 1)(lambda: flush(aggB))
  plsc.subcore_barrier()
  zero_acc()
  plsc.subcore_barrier()
  pl.when(c == 0)(lambda: gather_pass(hC))
  if do_deg:
    pl.when(c == 1)(deg_pass)
  plsc.subcore_barrier()
  pl.when(c == 0)(lambda: flush(aggC))
  if do_deg:
    pl.when(c == 1)(lambda: flush(degO))


def _make_sc(do_deg):
  outs = [jax.ShapeDtypeStruct((NPAD, 16), jnp.float32)] * (4 if do_deg else 3)
  return pl.kernel(
      functools.partial(_sc_body, do_deg),
      out_type=outs,
      mesh=plsc.VectorSubcoreMesh(core_axis_name="c", subcore_axis_name="s"),
      compiler_params=pltpu.CompilerParams(use_tc_tiling_on_sc=False),
      scratch_types=[
          pltpu.VMEM_SHARED((NPAD, 16), jnp.float32),
          pltpu.VMEM((IDXD, CHUNK), jnp.int32),
          pltpu.VMEM((IDXD, 1, CHUNK), jnp.int32),
          pltpu.VMEM((DEPTH, CHUNK, 16), jnp.float32),
          pltpu.VMEM((FC, 16), jnp.float32),
      ] + [pltpu.SemaphoreType.DMA] * (IDXD + 2 * DEPTH),
  )


def _k1_body(x_ref, xd_ref, xst_ref, emb_ref, h0_ref, hA_ref, hB_ref, hC_ref):
  xst = xst_ref[...]
  oh = (lax.broadcasted_iota(jnp.int32, (BN, 256), 1) == xst).astype(jnp.float32)
  emb = jnp.dot(oh, emb_ref[...], preferred_element_type=jnp.float32)
  h = jnp.concatenate(
      [x_ref[...], xd_ref[...], emb, jnp.zeros((BN, 2), jnp.float32)], axis=1)
  h0_ref[...] = h
  hA_ref[...] = h[:, 0:16]
  hB_ref[...] = h[:, 16:32]
  hC_ref[...] = h[:, 32:48]


def _combine(aA, aB, aC, dg, h, WlT, bl, WrT, g, b):
  agg = jnp.concatenate([aA[...], aB[...], aC[...]], axis=1)
  invd = 1.0 / jnp.maximum(dg[...][:, 0:1], 1.0)
  z = (jnp.dot(agg * invd, WlT[...], preferred_element_type=jnp.float32)
       + bl[...]
       + jnp.dot(h[...], WrT[...], preferred_element_type=jnp.float32))
  mu = jnp.mean(z, axis=1, keepdims=True)
  var = jnp.mean((z - mu) ** 2, axis=1, keepdims=True)
  zn = (z - mu) * lax.rsqrt(var + 1e-5) * g[...] + b[...]
  return jnp.maximum(zn, 0.0)


def _k2_body(aA, aB, aC, dg, h0, WlT, bl, WrT, g, b,
             h1_ref, oA_ref, oB_ref, oC_ref):
  h1 = _combine(aA, aB, aC, dg, h0, WlT, bl, WrT, g, b)
  h1_ref[...] = h1
  oA_ref[...] = h1[:, 0:16]
  oB_ref[...] = h1[:, 16:32]
  oC_ref[...] = h1[:, 32:48]


def _k3_body(aA, aB, aC, dg, h1, WlT, bl, WrT, g, b, bt_ref,
             W1T, b1, W2T, b2, W3T, b3, out_ref, ssum, smax, scnt):
  i = pl.program_id(0)

  @pl.when(i == 0)
  def _init():
    ssum[...] = jnp.zeros((NUM_GRAPHS, H), jnp.float32)
    smax[...] = jnp.full((NUM_GRAPHS, H), -jnp.inf, jnp.float32)
    scnt[...] = jnp.zeros((NUM_GRAPHS, 1), jnp.float32)

  h2 = _combine(aA, aB, aC, dg, h1, WlT, bl, WrT, g, b)
  bt = bt_ref[...]
  m = bt == lax.broadcasted_iota(jnp.int32, (BN, NUM_GRAPHS), 1)
  mf = m.astype(jnp.float32)
  ssum[...] += lax.dot_general(mf, h2, (((0,), (0,)), ((), ())),
                               preferred_element_type=jnp.float32)
  scnt[...] += jnp.sum(mf, axis=0).reshape(NUM_GRAPHS, 1)
  for gi in range(NUM_GRAPHS):
    v = jnp.max(jnp.where(m[:, gi:gi + 1], h2, -jnp.inf), axis=0)
    smax[gi:gi + 1, :] = jnp.maximum(smax[gi:gi + 1, :], v.reshape(1, H))

  @pl.when(i == GRID - 1)
  def _fin():
    mean = ssum[...] / jnp.maximum(scnt[...], 1.0)
    zz = jnp.concatenate([mean, smax[...]], axis=1)
    zz = jnp.maximum(jnp.dot(zz, W1T[...], preferred_element_type=jnp.float32) + b1[...], 0.0)
    zz = jnp.maximum(jnp.dot(zz, W2T[...], preferred_element_type=jnp.float32) + b2[...], 0.0)
    zz = jnp.maximum(jnp.dot(zz, W3T[...], preferred_element_type=jnp.float32) + b3[...], 0.0)
    out_ref[...] = zz


def _row_spec(w):
  return pl.BlockSpec((BN, w), lambda i: (i, 0))


def _full_spec(shape):
  return pl.BlockSpec(shape, lambda i: tuple(0 for _ in shape))


def kernel(x, edge_index, edge_attr, xdims, xsttype, batch, st_emb,
           Wl1, bl1, Wr1, g1, beta1, Wl2, bl2, Wr2, g2, beta2,
           Wfc1, bfc1, Wfc2, bfc2, Wfc3, bfc3):
  f32 = jnp.float32
  srcp = jnp.concatenate([edge_index[0], jnp.zeros((EPAD - E,), jnp.int32)])
  dstp = jnp.concatenate([edge_index[1], jnp.full((EPAD - E,), N, jnp.int32)])
  dst2d = dstp.reshape(EPAD // CHUNK, CHUNK)
  Wl1T = jnp.pad(Wl1, ((0, 0), (0, 2))).T
  Wr1T = jnp.pad(Wr1, ((0, 0), (0, 2))).T
  Wl2T = Wl2.T
  Wr2T = Wr2.T
  row = lambda v: v.reshape(1, -1)

  k1 = pl.pallas_call(
      _k1_body,
      grid=(GRID,),
      in_specs=[_row_spec(32), _row_spec(2), _row_spec(1), _full_spec((256, 12))],
      out_specs=[_row_spec(48), _row_spec(16), _row_spec(16), _row_spec(16)],
      out_shape=[jax.ShapeDtypeStruct((N, 48), f32)] +
                [jax.ShapeDtypeStruct((N, 16), f32)] * 3,
  )
  h0, hA, hB, hC = k1(x, xdims, xsttype.reshape(N, 1), st_emb)

  sc1 = _make_sc(True)
  aggA, aggB, aggC, deg = sc1(hA, hB, hC, srcp, dst2d)

  wspec = [_full_spec((48, 48)), _full_spec((1, 48)), _full_spec((48, 48)),
           _full_spec((1, 48)), _full_spec((1, 48))]
  k2 = pl.pallas_call(
      _k2_body,
      grid=(GRID,),
      in_specs=[_row_spec(16)] * 3 + [_row_spec(16), _row_spec(48)] + wspec,
      out_specs=[_row_spec(48), _row_spec(16), _row_spec(16), _row_spec(16)],
      out_shape=[jax.ShapeDtypeStruct((N, 48), f32)] +
                [jax.ShapeDtypeStruct((N, 16), f32)] * 3,
  )
  h1, h1A, h1B, h1C = k2(aggA, aggB, aggC, deg, h0,
                         Wl1T, row(bl1), Wr1T, row(g1), row(beta1))

  sc2 = _make_sc(False)
  agg2A, agg2B, agg2C = sc2(h1A, h1B, h1C, srcp, dst2d)

  k3 = pl.pallas_call(
      _k3_body,
      grid=(GRID,),
      in_specs=([_row_spec(16)] * 3 + [_row_spec(16), _row_spec(48)] + wspec +
                [_row_spec(1),
                 _full_spec((96, 50)), _full_spec((1, 50)),
                 _full_spec((50, 50)), _full_spec((1, 50)),
                 _full_spec((50, 10)), _full_spec((1, 10))]),
      out_specs=pl.BlockSpec((NUM_GRAPHS, 10), lambda i: (0, 0)),
      out_shape=jax.ShapeDtypeStruct((NUM_GRAPHS, 10), f32),
      scratch_shapes=[pltpu.VMEM((NUM_GRAPHS, H), f32),
                      pltpu.VMEM((NUM_GRAPHS, H), f32),
                      pltpu.VMEM((NUM_GRAPHS, 1), f32)],
  )
  out = k3(agg2A, agg2B, agg2C, deg, h1,
           Wl2T, row(bl2), Wr2T, row(g2), row(beta2), batch.reshape(N, 1),
           Wfc1.T, row(bfc1), Wfc2.T, row(bfc2), Wfc3.T, row(bfc3))
  return out
